# baseline (speedup 1.0000x reference)
import functools

import jax
import jax.numpy as jnp
from jax import lax
from jax.experimental import pallas as pl
from jax.experimental.pallas import tpu as pltpu
from jax.experimental.pallas import tpu_sc as plsc

UN = 50000
IN_ = 50000
DD = 32
EE = 800000
LL = 2

NC = 2
NS = 16
CH = 80
EPT = EE // NS
NCHUNK = EPT // CH
UNP = 50176
RPT = UNP // NS
ZR = 112
NP = 50176
HPT = NP // NS

NB = 5
NG = NCHUNK // NB

_mesh = plsc.VectorSubcoreMesh(core_axis_name="c", subcore_axis_name="s")


@functools.partial(
    pl.kernel,
    out_type=jax.ShapeDtypeStruct((2 * 2 * NP,), jnp.float32),
    mesh=_mesh,
    compiler_params=pltpu.CompilerParams(use_tc_tiling_on_sc=False),
    scratch_types=(
        [pltpu.VMEM((CH,), jnp.int32) for _ in range(NB)]
        + [
            pltpu.VMEM((CH,), jnp.float32),
            pltpu.VMEM((HPT,), jnp.float32),
            pltpu.VMEM_SHARED((NP,), jnp.float32),
            pltpu.VMEM_SHARED((NP,), jnp.float32),
        ]
        + [pltpu.SemaphoreType.DMA] * (2 * NB)
    ),
)
def _hist_kernel(rows1f, rows2f, out, *scr):
    idx_v = scr[0:NB]
    ones_v, zer_v, hacc0, hacc1 = scr[NB:NB + 4]
    semi = scr[NB + 4:2 * NB + 4]
    sems = scr[2 * NB + 4:3 * NB + 4]
    cid = lax.axis_index("c")
    sid = lax.axis_index("s")
    for i in range(CH // 16):
        ones_v[pl.ds(i * 16, 16)] = jnp.ones((16,), jnp.float32)

    def zinit(i, _):
        zer_v[pl.ds(i * 16, 16)] = jnp.zeros((16,), jnp.float32)
        return 0

    lax.fori_loop(0, HPT // 16, zinit, 0)
    pltpu.sync_copy(zer_v, hacc0.at[pl.ds(sid * HPT, HPT)])
    pltpu.sync_copy(zer_v, hacc1.at[pl.ds(sid * HPT, HPT)])
    plsc.subcore_barrier()

    for rowsf, hacc in ((rows1f, hacc0), (rows2f, hacc1)):
        ebase = cid * EE + sid * EPT

        def fire_idx(c, s):
            pltpu.async_copy(rowsf.at[pl.ds(ebase + c * CH, CH)],
                             idx_v[s], semi[s])

        def wait_idx(c, s):
            pltpu.make_async_copy(rowsf.at[pl.ds(ebase + c * CH, CH)],
                                  idx_v[s], semi[s]).wait()

        def fire_scatter(s):
            pltpu.async_copy(ones_v, hacc.at[idx_v[s]], sems[s], add=True)

        def wait_scatter(s):
            pltpu.make_async_copy(ones_v, hacc.at[idx_v[s]], sems[s]).wait()

        fire_idx(0, 0)
        fire_idx(1, 1)

        def group(g, _):
            for b in range(NB):
                c = g * NB + b
                s2 = (b + 2) % NB
                wait_idx(c, b)
                fire_scatter(b)
                pl.when(c >= 3)(lambda s2=s2: wait_scatter(s2))
                pl.when(c + 2 < NCHUNK)(
                    lambda c=c, s2=s2: fire_idx(c + 2, s2))
            return 0

        lax.fori_loop(0, NG, group, 0)
        wait_scatter((NCHUNK - 3) % NB)
        wait_scatter((NCHUNK - 2) % NB)
        wait_scatter((NCHUNK - 1) % NB)

    plsc.subcore_barrier()
    for p, hacc in enumerate((hacc0, hacc1)):
        pltpu.sync_copy(hacc.at[pl.ds(sid * HPT, HPT)], zer_v)
        pltpu.sync_copy(
            zer_v,
            out.at[pl.ds(p * 2 * NP + cid * NP + sid * HPT, HPT)],
        )


@functools.partial(
    pl.kernel,
    out_type=(
        jax.ShapeDtypeStruct((2 * UNP, DD), jnp.float32),
        jax.ShapeDtypeStruct((2 * UNP, DD), jnp.float32),
    ),
    mesh=_mesh,
    compiler_params=pltpu.CompilerParams(use_tc_tiling_on_sc=False),
    scratch_types=(
        [pltpu.VMEM((CH,), jnp.int32) for _ in range(NB)]
        + [pltpu.VMEM((CH,), jnp.int32) for _ in range(NB)]
        + [pltpu.VMEM((CH, DD), jnp.float32) for _ in range(NB)]
        + [pltpu.VMEM((ZR, DD), jnp.float32)] * 2
        + [pltpu.VMEM_SHARED((UNP, DD), jnp.float32)]
        + [pltpu.SemaphoreType.DMA] * (3 * NB)
    ),
)
def _spmm_kernel(rows1f, cols1f, rows2f, cols2f, tab1, tab2,
                 out1, out2, *scr):
    rows_v = scr[0:NB]
    cols_v = scr[NB:2 * NB]
    gath_v = scr[2 * NB:3 * NB]
    zer_v, stage_v, acc = scr[3 * NB:3 * NB + 3]
    semi = scr[3 * NB + 3:4 * NB + 3]
    semg = scr[4 * NB + 3:5 * NB + 3]
    sems = scr[5 * NB + 3:6 * NB + 3]
    cid = lax.axis_index("c")
    sid = lax.axis_index("s")

    z16 = jnp.zeros((16,), jnp.float32)
    for r in range(ZR):
        zer_v[r, pl.ds(0, 16)] = z16
        zer_v[r, pl.ds(16, 16)] = z16

    for rowsf, colsf, tab, out in ((rows1f, cols1f, tab1, out1),
                                   (rows2f, cols2f, tab2, out2)):
        def zero_body(j, _):
            pltpu.sync_copy(zer_v, acc.at[pl.ds(sid * RPT + j * ZR, ZR)])
            return 0

        lax.fori_loop(0, RPT // ZR, zero_body, 0)
        plsc.subcore_barrier()

        ebase = cid * EE + sid * EPT

        def fire_idx(c, s):
            pltpu.async_copy(rowsf.at[pl.ds(ebase + c * CH, CH)],
                             rows_v[s], semi[s])
            pltpu.async_copy(colsf.at[pl.ds(ebase + c * CH, CH)],
                             cols_v[s], semi[s])

        def wait_idx(c, s):
            pltpu.make_async_copy(rowsf.at[pl.ds(ebase + c * CH, CH)],
                                  rows_v[s], semi[s]).wait()
            pltpu.make_async_copy(colsf.at[pl.ds(ebase + c * CH, CH)],
                                  cols_v[s], semi[s]).wait()

        def fire_gather(s):
            pltpu.async_copy(tab.at[cols_v[s]], gath_v[s], semg[s])

        def wait_gather(s):
            pltpu.make_async_copy(tab.at[cols_v[s]], gath_v[s],
                                  semg[s]).wait()

        def fire_scatter(s):
            pltpu.async_copy(gath_v[s], acc.at[rows_v[s]], sems[s], add=True)

        def wait_scatter(s):
            pltpu.make_async_copy(gath_v[s], acc.at[rows_v[s]],
                                  sems[s]).wait()

        fire_idx(0, 0)
        fire_idx(1, 1)
        wait_idx(0, 0)
        fire_gather(0)

        def group(g, _):
            for b in range(NB):
                c = g * NB + b
                s1 = (b + 1) % NB
                s2 = (b + 2) % NB
                wait_gather(b)
                fire_scatter(b)
                pl.when(c >= 3)(lambda s2=s2: wait_scatter(s2))
                pl.when(c + 2 < NCHUNK)(
                    lambda c=c, s2=s2: fire_idx(c + 2, s2))
                pl.when(c + 1 < NCHUNK)(
                    lambda c=c, s1=s1: (wait_idx(c + 1, s1), fire_gather(s1))
                    and None)
            return 0

        lax.fori_loop(0, NG, group, 0)
        wait_scatter((NCHUNK - 3) % NB)
        wait_scatter((NCHUNK - 2) % NB)
        wait_scatter((NCHUNK - 1) % NB)
        plsc.subcore_barrier()

        def out_body(j, _):
            pltpu.sync_copy(acc.at[pl.ds(sid * RPT + j * ZR, ZR)], stage_v)
            pltpu.sync_copy(stage_v,
                            out.at[pl.ds(cid * UNP + sid * RPT + j * ZR, ZR)])
            return 0

        lax.fori_loop(0, RPT // ZR, out_body, 0)
        plsc.subcore_barrier()


BLK = 2000
NBLK = UN // BLK


def _dinv(deg):
    return jnp.where(deg > 0, lax.rsqrt(jnp.maximum(deg, 1e-12)), 0.0)


def _l2n(x):
    nrm = jnp.sqrt(jnp.sum(x * x, axis=-1, keepdims=True))
    return x / jnp.maximum(nrm, 1e-12)


def _prep_body(ue, ie, wu, bu, wi, bi, huu, hii, hbu, hbi,
               tab1, tab2, gu_o, gi_o):
    duu = _dinv(huu[...])
    dii = _dinv(hii[...])
    dbu = _dinv(hbu[...])
    dbi = _dinv(hbi[...])
    gu = ue[...] * jax.nn.softmax(ue[...] @ wu[...] + bu[...], axis=1)
    gi = ie[...] * jax.nn.softmax(ie[...] @ wi[...] + bi[...], axis=1)
    tab1[0] = duu * gu
    tab1[1] = dii * gi
    tab2[0] = dbi * gi
    tab2[1] = dbu * gu
    gu_o[...] = gu
    gi_o[...] = gi


def _combine_body(last, o1, o2, huu, hii, hbu, hbi, up, ip, *outs):
    duu = _dinv(huu[...])
    dii = _dinv(hii[...])
    dbu = _dinv(hbu[...])
    dbi = _dinv(hbi[...])
    ue = (duu * o1[0] + dbu * o2[0]) * 0.5
    ie = (dii * o1[1] + dbi * o2[1]) * 0.5
    ua = up[...] + _l2n(ue)
    ia = ip[...] + _l2n(ie)
    if last:
        (final,) = outs
        final[0] = ua
        final[1] = ia
    else:
        tab1, tab2, ua_o, ia_o = outs
        tab1[0] = duu * ue
        tab1[1] = dii * ie
        tab2[0] = dbi * ie
        tab2[1] = dbu * ue
        ua_o[...] = ua
        ia_o[...] = ia


_row_spec = pl.BlockSpec((BLK, DD), lambda i: (i, 0))
_stk_spec = pl.BlockSpec((2, BLK, DD), lambda i: (0, i, 0))
_w_spec = pl.BlockSpec((DD, DD), lambda i: (0, 0))
_b_spec = pl.BlockSpec((1, DD), lambda i: (0, 0))
_c_spec = pl.BlockSpec((BLK, 1), lambda i: (i, 0))

_f32 = jnp.float32


def _prep_call(ue, ie, wu, bu, wi, bi, hs):
    return pl.pallas_call(
        _prep_body,
        grid=(NBLK,),
        in_specs=[_row_spec, _row_spec, _w_spec, _b_spec, _w_spec, _b_spec,
                  _c_spec, _c_spec, _c_spec, _c_spec],
        out_specs=[_stk_spec, _stk_spec, _row_spec, _row_spec],
        out_shape=[
            jax.ShapeDtypeStruct((2, UN, DD), _f32),
            jax.ShapeDtypeStruct((2, UN, DD), _f32),
            jax.ShapeDtypeStruct((UN, DD), _f32),
            jax.ShapeDtypeStruct((UN, DD), _f32),
        ],
    )(ue, ie, wu, bu, wi, bi, *hs)


def _combine_call(last, o1, o2, hs, up, ip):
    if last:
        out_specs = [_stk_spec]
        out_shape = [jax.ShapeDtypeStruct((2, UN, DD), _f32)]
    else:
        out_specs = [_stk_spec, _stk_spec, _row_spec, _row_spec]
        out_shape = [
            jax.ShapeDtypeStruct((2, UN, DD), _f32),
            jax.ShapeDtypeStruct((2, UN, DD), _f32),
            jax.ShapeDtypeStruct((UN, DD), _f32),
            jax.ShapeDtypeStruct((UN, DD), _f32),
        ]
    return pl.pallas_call(
        functools.partial(_combine_body, last),
        grid=(NBLK,),
        in_specs=[_stk_spec, _stk_spec, _c_spec, _c_spec, _c_spec, _c_spec,
                  _row_spec, _row_spec],
        out_specs=out_specs,
        out_shape=out_shape,
    )(o1, o2, *hs, up, ip)


def kernel(user_emb, item_emb, gating_weightu, gating_weightub,
           gating_weighti, gating_weightib,
           uu_rows, uu_cols, ii_rows, ii_cols, ui_rows, ui_cols):
    b_u = ui_rows[:EE]
    b_i = ui_cols[:EE] - UN

    off = jnp.int32(UN)
    rows1f = jnp.concatenate([uu_rows, ii_rows])
    cols1f = jnp.concatenate([uu_cols, ii_cols + off])
    rows2f = jnp.concatenate([b_u, b_i])
    cols2f = jnp.concatenate([b_i, b_u + off])

    hflat = _hist_kernel(rows1f, rows2f)
    h4 = hflat.reshape(4, NP)
    hs = tuple(h4[k].reshape(NP, 1) for k in range(4))

    tab1, tab2, ua, ia = _prep_call(
        user_emb, item_emb, gating_weightu, gating_weightub,
        gating_weighti, gating_weightib, hs)

    t1 = tab1.reshape(2 * UN, DD)
    t2 = tab2.reshape(2 * UN, DD)
    final = None
    for layer in range(LL):
        o1, o2 = _spmm_kernel(rows1f, cols1f, rows2f, cols2f, t1, t2)
        o1 = o1.reshape(2, UNP, DD)
        o2 = o2.reshape(2, UNP, DD)
        if layer + 1 < LL:
            tab1, tab2, ua, ia = _combine_call(False, o1, o2, hs, ua, ia)
            t1 = tab1.reshape(2 * UN, DD)
            t2 = tab2.reshape(2 * UN, DD)
        else:
            (final,) = _combine_call(True, o1, o2, hs, ua, ia)
    return final.reshape(2 * UN, DD)


# speedup vs baseline: 2.0291x; 2.0291x over previous
import functools

import jax
import jax.numpy as jnp
from jax import lax
from jax.experimental import pallas as pl
from jax.experimental.pallas import tpu as pltpu
from jax.experimental.pallas import tpu_sc as plsc

UN = 50000
IN_ = 50000
DD = 32
EE = 800000
LL = 2

NC = 2
NS = 16
CH = 400
EPT = EE // NS
NCHUNK = EPT // CH
UNP = 50176
RPT = UNP // NS
ZR = 32
NP = 50176
HPT = NP // NS

NI = 4
NGA = 2
GRP = 4
NGRP = (NCHUNK - 1) // GRP
NB = 5

CHH = 2000
NCH_H = EPT // CHH
NGH = NCH_H // NB

_mesh = plsc.VectorSubcoreMesh(core_axis_name="c", subcore_axis_name="s")


@functools.partial(
    pl.kernel,
    out_type=jax.ShapeDtypeStruct((2 * 2 * NP,), jnp.float32),
    mesh=_mesh,
    compiler_params=pltpu.CompilerParams(use_tc_tiling_on_sc=False),
    scratch_types=(
        [pltpu.VMEM((CHH,), jnp.int32) for _ in range(NB)]
        + [
            pltpu.VMEM((CHH,), jnp.float32),
            pltpu.VMEM((HPT,), jnp.float32),
            pltpu.VMEM_SHARED((NP,), jnp.float32),
            pltpu.VMEM_SHARED((NP,), jnp.float32),
        ]
        + [pltpu.SemaphoreType.DMA] * (2 * NB)
    ),
)
def _hist_kernel(rows1f, rows2f, out, *scr):
    idx_v = scr[0:NB]
    ones_v, zer_v, hacc0, hacc1 = scr[NB:NB + 4]
    semi = scr[NB + 4:2 * NB + 4]
    sems = scr[2 * NB + 4:3 * NB + 4]
    cid = lax.axis_index("c")
    sid = lax.axis_index("s")
    for i in range(CHH // 16):
        ones_v[pl.ds(i * 16, 16)] = jnp.ones((16,), jnp.float32)

    def zinit(i, _):
        zer_v[pl.ds(i * 16, 16)] = jnp.zeros((16,), jnp.float32)
        return 0

    lax.fori_loop(0, HPT // 16, zinit, 0)
    pltpu.sync_copy(zer_v, hacc0.at[pl.ds(sid * HPT, HPT)])
    pltpu.sync_copy(zer_v, hacc1.at[pl.ds(sid * HPT, HPT)])
    plsc.subcore_barrier()

    for rowsf, hacc in ((rows1f, hacc0), (rows2f, hacc1)):
        ebase = cid * EE + sid * EPT

        def fire_idx(c, s):
            pltpu.async_copy(rowsf.at[pl.ds(ebase + c * CHH, CHH)],
                             idx_v[s], semi[s])

        def wait_idx(c, s):
            pltpu.make_async_copy(rowsf.at[pl.ds(ebase + c * CHH, CHH)],
                                  idx_v[s], semi[s]).wait()

        def fire_scatter(s):
            pltpu.async_copy(ones_v, hacc.at[idx_v[s]], sems[s], add=True)

        def wait_scatter(s):
            pltpu.make_async_copy(ones_v, hacc.at[idx_v[s]], sems[s]).wait()

        fire_idx(0, 0)
        fire_idx(1, 1)

        def group(g, _):
            for b in range(NB):
                c = g * NB + b
                s2 = (b + 2) % NB
                wait_idx(c, b)
                fire_scatter(b)
                pl.when(c >= 3)(lambda s2=s2: wait_scatter(s2))
                pl.when(c + 2 < NCH_H)(
                    lambda c=c, s2=s2: fire_idx(c + 2, s2))
            return 0

        lax.fori_loop(0, NGH, group, 0)
        wait_scatter((NCH_H - 3) % NB)
        wait_scatter((NCH_H - 2) % NB)
        wait_scatter((NCH_H - 1) % NB)

    plsc.subcore_barrier()
    for p, hacc in enumerate((hacc0, hacc1)):
        pltpu.sync_copy(hacc.at[pl.ds(sid * HPT, HPT)], zer_v)
        pltpu.sync_copy(
            zer_v,
            out.at[pl.ds(p * 2 * NP + cid * NP + sid * HPT, HPT)],
        )


@functools.partial(
    pl.kernel,
    out_type=(
        jax.ShapeDtypeStruct((2 * UNP, DD), jnp.float32),
        jax.ShapeDtypeStruct((2 * UNP, DD), jnp.float32),
    ),
    mesh=_mesh,
    compiler_params=pltpu.CompilerParams(use_tc_tiling_on_sc=False),
    scratch_types=(
        [pltpu.VMEM((CH,), jnp.int32) for _ in range(NI)]
        + [pltpu.VMEM((CH,), jnp.int32) for _ in range(NI)]
        + [pltpu.VMEM((CH, DD), jnp.float32) for _ in range(NGA)]
        + [pltpu.VMEM((ZR, DD), jnp.float32)]
        + [pltpu.VMEM_SHARED((UNP, DD), jnp.float32)]
        + [pltpu.SemaphoreType.DMA] * (NI + 2 * NGA + 1)
    ),
)
def _spmm_kernel(rows1f, cols1f, rows2f, cols2f, tab1, tab2,
                 out1, out2, *scr):
    rows_v = scr[0:NI]
    cols_v = scr[NI:2 * NI]
    gath_v = scr[2 * NI:2 * NI + NGA]
    zer_v, acc = scr[2 * NI + NGA:2 * NI + NGA + 2]
    base = 2 * NI + NGA + 2
    semi = scr[base:base + NI]
    semg = scr[base + NI:base + NI + NGA]
    sems = scr[base + NI + NGA:base + NI + 2 * NGA]
    semz = scr[base + NI + 2 * NGA]
    cid = lax.axis_index("c")
    sid = lax.axis_index("s")

    z16 = jnp.zeros((16,), jnp.float32)
    for r in range(ZR):
        zer_v[r, pl.ds(0, 16)] = z16
        zer_v[r, pl.ds(16, 16)] = z16

    NZ = RPT // ZR
    LAG = 16

    def _zfire(j):
        pltpu.async_copy(zer_v, acc.at[pl.ds(sid * RPT + j * ZR, ZR)], semz)

    def _zwait():
        pltpu.make_async_copy(zer_v, acc.at[pl.ds(sid * RPT, ZR)],
                              semz).wait()

    def zero_step(j, _):
        pl.when(j < NZ)(lambda: _zfire(j))
        pl.when(j >= LAG)(_zwait)
        return 0

    lax.fori_loop(0, NZ + LAG, zero_step, 0)
    plsc.subcore_barrier()

    for rowsf, colsf, tab, out in ((rows1f, cols1f, tab1, out1),
                                   (rows2f, cols2f, tab2, out2)):
        ebase = cid * EE + sid * EPT

        def fire_idx(c, si):
            pltpu.async_copy(rowsf.at[pl.ds(ebase + c * CH, CH)],
                             rows_v[si], semi[si])
            pltpu.async_copy(colsf.at[pl.ds(ebase + c * CH, CH)],
                             cols_v[si], semi[si])

        def wait_idx(c, si):
            pltpu.make_async_copy(rowsf.at[pl.ds(ebase + c * CH, CH)],
                                  rows_v[si], semi[si]).wait()
            pltpu.make_async_copy(colsf.at[pl.ds(ebase + c * CH, CH)],
                                  cols_v[si], semi[si]).wait()

        def fire_gather(si, sg):
            pltpu.async_copy(tab.at[cols_v[si]], gath_v[sg], semg[sg])

        def wait_gather(si, sg):
            pltpu.make_async_copy(tab.at[cols_v[si]], gath_v[sg],
                                  semg[sg]).wait()

        def fire_scatter(si, sg):
            pltpu.async_copy(gath_v[sg], acc.at[rows_v[si]], sems[sg],
                             add=True)

        def wait_scatter(si, sg):
            pltpu.make_async_copy(gath_v[sg], acc.at[rows_v[si]],
                                  sems[sg]).wait()

        def chunk_step(c, b, guard):
            si = b % NI
            sg = b % NGA
            sip = (b - 1) % NI
            sgp = (b - 1) % NGA
            sin = (b + 1) % NI
            sgn = (b + 1) % NGA
            si2 = (b + 2) % NI
            wait_gather(si, sg)
            fire_scatter(si, sg)
            if guard:
                pl.when(c >= 1)(lambda: wait_scatter(sip, sgp))
                pl.when(c + 1 < NCHUNK)(lambda: wait_idx(c + 1, sin))
                pl.when(c + 1 < NCHUNK)(lambda: fire_gather(sin, sgn))
                pl.when(c + 2 < NCHUNK)(lambda: fire_idx(c + 2, si2))
            else:
                if c >= 1:
                    wait_scatter(sip, sgp)
                if c + 1 < NCHUNK:
                    wait_idx(c + 1, sin)
                    fire_gather(sin, sgn)
                if c + 2 < NCHUNK:
                    fire_idx(c + 2, si2)

        fire_idx(0, 0)
        fire_idx(1, 1)
        wait_idx(0, 0)
        fire_gather(0, 0)

        def group(g, _):
            for b in range(GRP):
                chunk_step(g * GRP + b, b, True)
            return 0

        lax.fori_loop(0, NGRP, group, 0)
        for c in range(NGRP * GRP, NCHUNK):
            chunk_step(c, c % GRP, False)
        wait_scatter((NCHUNK - 1) % NI, (NCHUNK - 1) % NGA)
        plsc.subcore_barrier()
        pltpu.sync_copy(acc.at[pl.ds(sid * RPT, RPT)],
                        out.at[pl.ds(cid * UNP + sid * RPT, RPT)])
        plsc.subcore_barrier()


BLK = 2000
NBLK = UN // BLK


def _dinv(deg):
    return jnp.where(deg > 0, lax.rsqrt(jnp.maximum(deg, 1e-12)), 0.0)


def _l2n(x):
    nrm = jnp.sqrt(jnp.sum(x * x, axis=-1, keepdims=True))
    return x / jnp.maximum(nrm, 1e-12)


def _prep_body(ue, ie, wu, bu, wi, bi, huu, hii, hbu, hbi,
               tab1, tab2, gu_o, gi_o):
    duu = _dinv(huu[...])
    dii = _dinv(hii[...])
    dbu = _dinv(hbu[...])
    dbi = _dinv(hbi[...])
    gu = ue[...] * jax.nn.softmax(ue[...] @ wu[...] + bu[...], axis=1)
    gi = ie[...] * jax.nn.softmax(ie[...] @ wi[...] + bi[...], axis=1)
    tab1[0] = duu * gu
    tab1[1] = dii * gi
    tab2[0] = dbi * gi
    tab2[1] = dbu * gu
    gu_o[...] = gu
    gi_o[...] = gi


def _combine_body(last, o1, o2, huu, hii, hbu, hbi, up, ip, *outs):
    duu = _dinv(huu[...])
    dii = _dinv(hii[...])
    dbu = _dinv(hbu[...])
    dbi = _dinv(hbi[...])
    ue = (duu * o1[0] + dbu * (o2[0] - o1[0])) * 0.5
    ie = (dii * o1[1] + dbi * (o2[1] - o1[1])) * 0.5
    ua = up[...] + _l2n(ue)
    ia = ip[...] + _l2n(ie)
    if last:
        (final,) = outs
        final[0] = ua
        final[1] = ia
    else:
        tab1, tab2, ua_o, ia_o = outs
        tab1[0] = duu * ue
        tab1[1] = dii * ie
        tab2[0] = dbi * ie
        tab2[1] = dbu * ue
        ua_o[...] = ua
        ia_o[...] = ia


_row_spec = pl.BlockSpec((BLK, DD), lambda i: (i, 0))
_stk_spec = pl.BlockSpec((2, BLK, DD), lambda i: (0, i, 0))
_w_spec = pl.BlockSpec((DD, DD), lambda i: (0, 0))
_b_spec = pl.BlockSpec((1, DD), lambda i: (0, 0))
_c_spec = pl.BlockSpec((BLK, 1), lambda i: (i, 0))

_f32 = jnp.float32


def _prep_call(ue, ie, wu, bu, wi, bi, hs):
    return pl.pallas_call(
        _prep_body,
        grid=(NBLK,),
        in_specs=[_row_spec, _row_spec, _w_spec, _b_spec, _w_spec, _b_spec,
                  _c_spec, _c_spec, _c_spec, _c_spec],
        out_specs=[_stk_spec, _stk_spec, _row_spec, _row_spec],
        out_shape=[
            jax.ShapeDtypeStruct((2, UN, DD), _f32),
            jax.ShapeDtypeStruct((2, UN, DD), _f32),
            jax.ShapeDtypeStruct((UN, DD), _f32),
            jax.ShapeDtypeStruct((UN, DD), _f32),
        ],
    )(ue, ie, wu, bu, wi, bi, *hs)


def _combine_call(last, o1, o2, hs, up, ip):
    if last:
        out_specs = [_stk_spec]
        out_shape = [jax.ShapeDtypeStruct((2, UN, DD), _f32)]
    else:
        out_specs = [_stk_spec, _stk_spec, _row_spec, _row_spec]
        out_shape = [
            jax.ShapeDtypeStruct((2, UN, DD), _f32),
            jax.ShapeDtypeStruct((2, UN, DD), _f32),
            jax.ShapeDtypeStruct((UN, DD), _f32),
            jax.ShapeDtypeStruct((UN, DD), _f32),
        ]
    return pl.pallas_call(
        functools.partial(_combine_body, last),
        grid=(NBLK,),
        in_specs=[_stk_spec, _stk_spec, _c_spec, _c_spec, _c_spec, _c_spec,
                  _row_spec, _row_spec],
        out_specs=out_specs,
        out_shape=out_shape,
    )(o1, o2, *hs, up, ip)


def kernel(user_emb, item_emb, gating_weightu, gating_weightub,
           gating_weighti, gating_weightib,
           uu_rows, uu_cols, ii_rows, ii_cols, ui_rows, ui_cols):
    b_u = ui_rows[:EE]
    b_i = ui_cols[:EE] - UN

    off = jnp.int32(UN)
    rows1f = jnp.concatenate([uu_rows, ii_rows])
    cols1f = jnp.concatenate([uu_cols, ii_cols + off])
    rows2f = jnp.concatenate([b_u, b_i])
    cols2f = jnp.concatenate([b_i, b_u + off])

    hflat = _hist_kernel(rows1f, rows2f)
    h4 = hflat.reshape(4, NP)
    hs = tuple(h4[k].reshape(NP, 1) for k in range(4))

    tab1, tab2, ua, ia = _prep_call(
        user_emb, item_emb, gating_weightu, gating_weightub,
        gating_weighti, gating_weightib, hs)

    t1 = tab1.reshape(2 * UN, DD)
    t2 = tab2.reshape(2 * UN, DD)
    final = None
    for layer in range(LL):
        o1, o2 = _spmm_kernel(rows1f, cols1f, rows2f, cols2f, t1, t2)
        o1 = o1.reshape(2, UNP, DD)
        o2 = o2.reshape(2, UNP, DD)
        if layer + 1 < LL:
            tab1, tab2, ua, ia = _combine_call(False, o1, o2, hs, ua, ia)
            t1 = tab1.reshape(2 * UN, DD)
            t2 = tab2.reshape(2 * UN, DD)
        else:
            (final,) = _combine_call(True, o1, o2, hs, ua, ia)
    return final.reshape(2 * UN, DD)
